# Initial kernel scaffold; baseline (speedup 1.0000x reference)
import functools

import jax
import jax.numpy as jnp
from jax import lax
from jax.experimental import pallas as pl
from jax.experimental.pallas import tpu as pltpu
from jax.experimental.pallas import tpu_sc as plsc

N = 10000
E = 320000
D = 128

NC = 2
NS = 16
CHUNK = 128
CPT = 80
E_PAD = NC * NS * CHUNK * CPT
N_PAD = 10240
SINK = 10200
RPT = N_PAD // NS

_MESH = dict(core_axis_name="c", subcore_axis_name="s")


def _sc_degree(dst_pad, ones_blk, zeros16):
    mesh = plsc.VectorSubcoreMesh(**_MESH)

    @functools.partial(
        pl.kernel,
        out_type=jax.ShapeDtypeStruct((NC * N_PAD, 16), jnp.float32),
        mesh=mesh,
        scratch_types=[
            pltpu.VMEM((CHUNK,), jnp.int32),
            pltpu.VMEM((CHUNK, 16), jnp.float32),
            pltpu.VMEM_SHARED((N_PAD, 16), jnp.float32),
        ],
    )
    def k(dst_hbm, ones_hbm, zeros_hbm, out_hbm, idx_v, ones_v, acc_sh):
        c = lax.axis_index("c")
        s = lax.axis_index("s")
        pltpu.sync_copy(zeros_hbm.at[pl.ds(s * RPT, RPT)],
                        acc_sh.at[pl.ds(s * RPT, RPT)])
        pltpu.sync_copy(ones_hbm, ones_v)
        plsc.subcore_barrier()
        tile_base = (c * NS + s) * (CHUNK * CPT)

        @pl.loop(0, CPT)
        def _(j):
            pltpu.sync_copy(dst_hbm.at[pl.ds(tile_base + j * CHUNK, CHUNK)],
                            idx_v)
            pltpu.sync_copy(ones_v, acc_sh.at[idx_v], add=True)

        plsc.subcore_barrier()
        pltpu.sync_copy(acc_sh.at[pl.ds(s * RPT, RPT)],
                        out_hbm.at[pl.ds(c * N_PAD + s * RPT, RPT)])

    return k(dst_pad, ones_blk, zeros16)


def _sc_aggregate(g, src_pad, dst_pad, zeros128):
    mesh = plsc.VectorSubcoreMesh(**_MESH)

    @functools.partial(
        pl.kernel,
        out_type=jax.ShapeDtypeStruct((NC * N_PAD, D), jnp.float32),
        mesh=mesh,
        scratch_types=[
            pltpu.VMEM((CHUNK,), jnp.int32),
            pltpu.VMEM((CHUNK,), jnp.int32),
            pltpu.VMEM((CHUNK, D), jnp.float32),
            pltpu.VMEM_SHARED((N_PAD, D), jnp.float32),
            pltpu.SemaphoreType.DMA,
        ],
    )
    def k(g_hbm, src_hbm, dst_hbm, zeros_hbm, out_hbm,
          sidx_v, didx_v, rows_v, acc_sh, sem):
        c = lax.axis_index("c")
        s = lax.axis_index("s")
        pltpu.sync_copy(zeros_hbm.at[pl.ds(s * RPT, RPT)],
                        acc_sh.at[pl.ds(s * RPT, RPT)])
        plsc.subcore_barrier()
        tile_base = (c * NS + s) * (CHUNK * CPT)

        @pl.loop(0, CPT)
        def _(j):
            base = tile_base + j * CHUNK
            pltpu.sync_copy(src_hbm.at[pl.ds(base, CHUNK)], sidx_v)
            pltpu.sync_copy(dst_hbm.at[pl.ds(base, CHUNK)], didx_v)
            pltpu.async_copy(g_hbm.at[sidx_v], rows_v, sem).wait()
            pltpu.sync_copy(rows_v, acc_sh.at[didx_v], add=True)

        plsc.subcore_barrier()
        pltpu.sync_copy(acc_sh.at[pl.ds(s * RPT, RPT)],
                        out_hbm.at[pl.ds(c * N_PAD + s * RPT, RPT)])

    return k(g, src_pad, dst_pad, zeros128)


def _tc_hw(x, W1, b1, Wc):
    BR = 1000

    def body(x_ref, w1_ref, b1_ref, wc_ref, o_ref):
        h = jnp.dot(x_ref[...], w1_ref[...],
                    preferred_element_type=jnp.float32,
                    precision=lax.Precision.HIGHEST)
        h = jnp.maximum(h + b1_ref[...], 0.0)
        o_ref[...] = jnp.dot(h, wc_ref[...],
                             preferred_element_type=jnp.float32,
                             precision=lax.Precision.HIGHEST)

    return pl.pallas_call(
        body,
        grid=(N // BR,),
        in_specs=[
            pl.BlockSpec((BR, D), lambda i: (i, 0)),
            pl.BlockSpec((D, D), lambda i: (0, 0)),
            pl.BlockSpec((1, D), lambda i: (0, 0)),
            pl.BlockSpec((D, D), lambda i: (0, 0)),
        ],
        out_specs=pl.BlockSpec((BR, D), lambda i: (i, 0)),
        out_shape=jax.ShapeDtypeStruct((N, D), jnp.float32),
    )(x, W1, b1.reshape(1, D), Wc)


def _tc_scale(deg_parts, hw):
    BR = 640

    def body(dp_ref, hw_ref, o_ref):
        i = pl.program_id(0)
        p = dp_ref[...]
        deg = 1.0 + p[0, :, 0:1] + p[1, :, 0:1]
        dinv = lax.rsqrt(deg)
        rows = i * BR + lax.broadcasted_iota(jnp.int32, (BR, 1), 0)
        o_ref[...] = jnp.where(rows < N, dinv * hw_ref[...], 0.0)

    return pl.pallas_call(
        body,
        grid=(N_PAD // BR,),
        in_specs=[
            pl.BlockSpec((NC, BR, 16), lambda i: (0, i, 0)),
            pl.BlockSpec((BR, D), lambda i: (i, 0)),
        ],
        out_specs=pl.BlockSpec((BR, D), lambda i: (i, 0)),
        out_shape=jax.ShapeDtypeStruct((N_PAD, D), jnp.float32),
    )(deg_parts, hw)


def _tc_final(agg_parts, g, deg_parts, bc, Wout, bout):
    BR = 640

    def body(agg_ref, g_ref, dp_ref, bc_ref, w_ref, bo_ref, o_ref):
        p = dp_ref[...]
        deg = 1.0 + p[0, :, 0:1] + p[1, :, 0:1]
        dinv = lax.rsqrt(deg)
        ssum = agg_ref[0] + agg_ref[1] + g_ref[...]
        h2 = jnp.maximum(dinv * ssum + bc_ref[...], 0.0)
        o_ref[...] = jnp.dot(h2, w_ref[...],
                             preferred_element_type=jnp.float32,
                             precision=lax.Precision.HIGHEST) + bo_ref[...]

    return pl.pallas_call(
        body,
        grid=(N_PAD // BR,),
        in_specs=[
            pl.BlockSpec((NC, BR, D), lambda i: (0, i, 0)),
            pl.BlockSpec((BR, D), lambda i: (i, 0)),
            pl.BlockSpec((NC, BR, 16), lambda i: (0, i, 0)),
            pl.BlockSpec((1, D), lambda i: (0, 0)),
            pl.BlockSpec((D, D), lambda i: (0, 0)),
            pl.BlockSpec((1, D), lambda i: (0, 0)),
        ],
        out_specs=pl.BlockSpec((BR, D), lambda i: (i, 0)),
        out_shape=jax.ShapeDtypeStruct((N, D), jnp.float32),
    )(agg_parts, g, deg_parts, bc.reshape(1, D), Wout, bout.reshape(1, D))


def kernel(x, edge_index, W1, b1, Wc, bc, Wout, bout):
    src = edge_index[0].astype(jnp.int32)
    dst = edge_index[1].astype(jnp.int32)
    pad = jnp.full((E_PAD - E,), SINK, jnp.int32)
    src_p = jnp.concatenate([src, pad])
    dst_p = jnp.concatenate([dst, pad])

    ones_blk = jnp.ones((CHUNK, 16), jnp.float32)
    zeros16 = jnp.zeros((N_PAD, 16), jnp.float32)
    zeros128 = jnp.zeros((N_PAD, D), jnp.float32)

    deg_parts = _sc_degree(dst_p, ones_blk, zeros16).reshape(NC, N_PAD, 16)
    hw = _tc_hw(x, W1, b1, Wc)
    g = _tc_scale(deg_parts, hw)
    agg_parts = _sc_aggregate(g, src_p, dst_p, zeros128).reshape(NC, N_PAD, D)
    return _tc_final(agg_parts, g, deg_parts, bc, Wout, bout)


# speedup vs baseline: 9.2600x; 9.2600x over previous
import functools

import jax
import jax.numpy as jnp
from jax import lax
from jax.experimental import pallas as pl
from jax.experimental.pallas import tpu as pltpu
from jax.experimental.pallas import tpu_sc as plsc

N = 10000
E = 320000
D = 128

NC = 2
NS = 16
CHUNK = 128
CPT = 80
E_PAD = NC * NS * CHUNK * CPT
N_PAD = 10240
SINK = 10200
RPT = N_PAD // NS

_MESH = dict(core_axis_name="c", subcore_axis_name="s")


def _sc_degree(dst_pad, ones_blk, zeros128):
    mesh = plsc.VectorSubcoreMesh(**_MESH)

    @functools.partial(
        pl.kernel,
        out_type=jax.ShapeDtypeStruct((NC * N_PAD, D), jnp.float32),
        mesh=mesh,
        scratch_types=[
            pltpu.VMEM((CHUNK,), jnp.int32),
            pltpu.VMEM((CHUNK, D), jnp.float32),
            pltpu.VMEM_SHARED((N_PAD, D), jnp.float32),
        ],
    )
    def k(dst_hbm, ones_hbm, zeros_hbm, out_hbm, idx_v, ones_v, acc_sh):
        c = lax.axis_index("c")
        s = lax.axis_index("s")
        pltpu.sync_copy(zeros_hbm.at[pl.ds(s * RPT, RPT)],
                        acc_sh.at[pl.ds(s * RPT, RPT)])
        pltpu.sync_copy(ones_hbm, ones_v)
        plsc.subcore_barrier()
        tile_base = (c * NS + s) * (CHUNK * CPT)

        @pl.loop(0, CPT)
        def _(j):
            pltpu.sync_copy(dst_hbm.at[pl.ds(tile_base + j * CHUNK, CHUNK)],
                            idx_v)
            pltpu.sync_copy(ones_v, acc_sh.at[idx_v], add=True)

        plsc.subcore_barrier()
        pltpu.sync_copy(acc_sh.at[pl.ds(s * RPT, RPT)],
                        out_hbm.at[pl.ds(c * N_PAD + s * RPT, RPT)])

    return k(dst_pad, ones_blk, zeros128)


def _sc_aggregate(g, src_pad, dst_pad, zeros128):
    mesh = plsc.VectorSubcoreMesh(**_MESH)

    @functools.partial(
        pl.kernel,
        out_type=jax.ShapeDtypeStruct((NC * N_PAD, D), jnp.float32),
        mesh=mesh,
        scratch_types=[
            pltpu.VMEM((CHUNK,), jnp.int32),
            pltpu.VMEM((CHUNK,), jnp.int32),
            pltpu.VMEM((CHUNK, D), jnp.float32),
            pltpu.VMEM_SHARED((N_PAD, D), jnp.float32),
            pltpu.SemaphoreType.DMA,
        ],
    )
    def k(g_hbm, src_hbm, dst_hbm, zeros_hbm, out_hbm,
          sidx_v, didx_v, rows_v, acc_sh, sem):
        c = lax.axis_index("c")
        s = lax.axis_index("s")
        pltpu.sync_copy(zeros_hbm.at[pl.ds(s * RPT, RPT)],
                        acc_sh.at[pl.ds(s * RPT, RPT)])
        plsc.subcore_barrier()
        tile_base = (c * NS + s) * (CHUNK * CPT)

        @pl.loop(0, CPT)
        def _(j):
            base = tile_base + j * CHUNK
            pltpu.sync_copy(src_hbm.at[pl.ds(base, CHUNK)], sidx_v)
            pltpu.sync_copy(dst_hbm.at[pl.ds(base, CHUNK)], didx_v)
            pltpu.async_copy(g_hbm.at[sidx_v], rows_v, sem).wait()
            pltpu.sync_copy(rows_v, acc_sh.at[didx_v], add=True)

        plsc.subcore_barrier()
        pltpu.sync_copy(acc_sh.at[pl.ds(s * RPT, RPT)],
                        out_hbm.at[pl.ds(c * N_PAD + s * RPT, RPT)])

    return k(g, src_pad, dst_pad, zeros128)


def _tc_hw(x, W1, b1, Wc):
    BR = 1000

    def body(x_ref, w1_ref, b1_ref, wc_ref, o_ref):
        h = jnp.dot(x_ref[...], w1_ref[...],
                    preferred_element_type=jnp.float32,
                    precision=lax.Precision.HIGHEST)
        h = jnp.maximum(h + b1_ref[...], 0.0)
        o_ref[...] = jnp.dot(h, wc_ref[...],
                             preferred_element_type=jnp.float32,
                             precision=lax.Precision.HIGHEST)

    return pl.pallas_call(
        body,
        grid=(N // BR,),
        in_specs=[
            pl.BlockSpec((BR, D), lambda i: (i, 0)),
            pl.BlockSpec((D, D), lambda i: (0, 0)),
            pl.BlockSpec((1, D), lambda i: (0, 0)),
            pl.BlockSpec((D, D), lambda i: (0, 0)),
        ],
        out_specs=pl.BlockSpec((BR, D), lambda i: (i, 0)),
        out_shape=jax.ShapeDtypeStruct((N, D), jnp.float32),
    )(x, W1, b1.reshape(1, D), Wc)


def _tc_scale(deg_parts, hw):
    BR = 640

    def body(dp_ref, hw_ref, o_ref):
        i = pl.program_id(0)
        p = dp_ref[...]
        deg = 1.0 + p[0, :, 0:1] + p[1, :, 0:1]
        dinv = lax.rsqrt(deg)
        rows = i * BR + lax.broadcasted_iota(jnp.int32, (BR, 1), 0)
        o_ref[...] = jnp.where(rows < N, dinv * hw_ref[...], 0.0)

    return pl.pallas_call(
        body,
        grid=(N_PAD // BR,),
        in_specs=[
            pl.BlockSpec((NC, BR, D), lambda i: (0, i, 0)),
            pl.BlockSpec((BR, D), lambda i: (i, 0)),
        ],
        out_specs=pl.BlockSpec((BR, D), lambda i: (i, 0)),
        out_shape=jax.ShapeDtypeStruct((N_PAD, D), jnp.float32),
    )(deg_parts, hw)


def _tc_final(agg_parts, g, deg_parts, bc, Wout, bout):
    BR = 640

    def body(agg_ref, g_ref, dp_ref, bc_ref, w_ref, bo_ref, o_ref):
        p = dp_ref[...]
        deg = 1.0 + p[0, :, 0:1] + p[1, :, 0:1]
        dinv = lax.rsqrt(deg)
        ssum = agg_ref[0] + agg_ref[1] + g_ref[...]
        h2 = jnp.maximum(dinv * ssum + bc_ref[...], 0.0)
        o_ref[...] = jnp.dot(h2, w_ref[...],
                             preferred_element_type=jnp.float32,
                             precision=lax.Precision.HIGHEST) + bo_ref[...]

    return pl.pallas_call(
        body,
        grid=(N_PAD // BR,),
        in_specs=[
            pl.BlockSpec((NC, BR, D), lambda i: (0, i, 0)),
            pl.BlockSpec((BR, D), lambda i: (i, 0)),
            pl.BlockSpec((NC, BR, D), lambda i: (0, i, 0)),
            pl.BlockSpec((1, D), lambda i: (0, 0)),
            pl.BlockSpec((D, D), lambda i: (0, 0)),
            pl.BlockSpec((1, D), lambda i: (0, 0)),
        ],
        out_specs=pl.BlockSpec((BR, D), lambda i: (i, 0)),
        out_shape=jax.ShapeDtypeStruct((N, D), jnp.float32),
    )(agg_parts, g, deg_parts, bc.reshape(1, D), Wout, bout.reshape(1, D))


def kernel(x, edge_index, W1, b1, Wc, bc, Wout, bout):
    src = edge_index[0].astype(jnp.int32)
    dst = edge_index[1].astype(jnp.int32)
    pad = jnp.full((E_PAD - E,), SINK, jnp.int32)
    src_p = jnp.concatenate([src, pad])
    dst_p = jnp.concatenate([dst, pad])

    ones_blk = jnp.ones((CHUNK, D), jnp.float32)
    zeros128 = jnp.zeros((N_PAD, D), jnp.float32)

    deg_parts = _sc_degree(dst_p, ones_blk, zeros128).reshape(NC, N_PAD, D)
    hw = _tc_hw(x, W1, b1, Wc)
    g = _tc_scale(deg_parts, hw)
    agg_parts = _sc_aggregate(g, src_p, dst_p, zeros128).reshape(NC, N_PAD, D)
    return _tc_final(agg_parts, g, deg_parts, bc, Wout, bout)


# speedup vs baseline: 30.9906x; 3.3467x over previous
import functools

import jax
import jax.numpy as jnp
from jax import lax
from jax.experimental import pallas as pl
from jax.experimental.pallas import tpu as pltpu
from jax.experimental.pallas import tpu_sc as plsc

N = 10000
E = 320000
D = 128

NC = 2
NS = 16
CHUNK = 128
CPT = 80
BCH = 20
E_PAD = NC * NS * CHUNK * CPT
N_PAD = 10240
SINK = 10200
RPT = N_PAD // NS

_MESH = dict(core_axis_name="c", subcore_axis_name="s")


def _sc_degree(dst_flat, ones_blk, zeros128):
    mesh = plsc.VectorSubcoreMesh(**_MESH)

    @functools.partial(
        pl.kernel,
        out_type=jax.ShapeDtypeStruct((NC * N_PAD, D), jnp.float32),
        mesh=mesh,
        scratch_types=[
            pltpu.VMEM((CHUNK,), jnp.int32),
            pltpu.VMEM((CHUNK,), jnp.int32),
            pltpu.VMEM((CHUNK, D), jnp.float32),
            pltpu.VMEM_SHARED((N_PAD, D), jnp.float32),
            pltpu.SemaphoreType.DMA,
            pltpu.SemaphoreType.DMA,
        ],
    )
    def k(dst_hbm, ones_hbm, zeros_hbm, out_hbm, idx0_v, idx1_v, ones_v,
          acc_sh, isem0, isem1):
        c = lax.axis_index("c")
        s = lax.axis_index("s")
        w = c * NS + s
        tile_base = w * (CHUNK * CPT)
        pltpu.sync_copy(zeros_hbm.at[pl.ds(s * RPT, RPT)],
                        acc_sh.at[pl.ds(s * RPT, RPT)])
        pltpu.sync_copy(ones_hbm, ones_v)
        plsc.subcore_barrier()

        pltpu.async_copy(dst_hbm.at[pl.ds(tile_base, CHUNK)], idx0_v, isem0)
        pltpu.async_copy(dst_hbm.at[pl.ds(tile_base + CHUNK, CHUNK)],
                         idx1_v, isem1)

        @pl.loop(0, CPT // 2)
        def _(kk):
            base = tile_base + 2 * kk * CHUNK
            pltpu.make_async_copy(dst_hbm.at[pl.ds(0, CHUNK)], idx0_v,
                                  isem0).wait()
            pltpu.sync_copy(ones_v, acc_sh.at[idx0_v], add=True)
            pltpu.async_copy(dst_hbm.at[pl.ds(base + 2 * CHUNK, CHUNK)],
                             idx0_v, isem0)
            pltpu.make_async_copy(dst_hbm.at[pl.ds(0, CHUNK)], idx1_v,
                                  isem1).wait()
            pltpu.sync_copy(ones_v, acc_sh.at[idx1_v], add=True)
            pltpu.async_copy(dst_hbm.at[pl.ds(base + 3 * CHUNK, CHUNK)],
                             idx1_v, isem1)

        pltpu.make_async_copy(dst_hbm.at[pl.ds(0, CHUNK)], idx0_v,
                              isem0).wait()
        pltpu.make_async_copy(dst_hbm.at[pl.ds(0, CHUNK)], idx1_v,
                              isem1).wait()

        plsc.subcore_barrier()
        pltpu.sync_copy(acc_sh.at[pl.ds(s * RPT, RPT)],
                        out_hbm.at[pl.ds(c * N_PAD + s * RPT, RPT)])

    return k(dst_flat, ones_blk, zeros128)


def _sc_aggregate(g, src_flat, dst_flat, zeros128):
    mesh = plsc.VectorSubcoreMesh(**_MESH)

    @functools.partial(
        pl.kernel,
        out_type=jax.ShapeDtypeStruct((NC * N_PAD, D), jnp.float32),
        mesh=mesh,
        scratch_types=(
            [pltpu.VMEM((CHUNK,), jnp.int32) for _ in range(8)]
            + [pltpu.VMEM((CHUNK, D), jnp.float32) for _ in range(2)]
            + [pltpu.VMEM_SHARED((N_PAD, D), jnp.float32)]
            + [pltpu.SemaphoreType.DMA for _ in range(6)]
        ),
    )
    def k(g_hbm, src_hbm, dst_hbm, zeros_hbm, out_hbm,
          s0, s1, s2, s3, d0, d1, d2, d3, rows0, rows1, acc_sh,
          gsem0, gsem1, i0, i1, i2, i3):
        sidx = [s0, s1, s2, s3]
        didx = [d0, d1, d2, d3]
        rows = [rows0, rows1]
        gsem = [gsem0, gsem1]
        isem = [i0, i1, i2, i3]
        c = lax.axis_index("c")
        s = lax.axis_index("s")
        w = c * NS + s
        tile_base = w * (CHUNK * CPT)
        pltpu.sync_copy(zeros_hbm.at[pl.ds(s * RPT, RPT)],
                        acc_sh.at[pl.ds(s * RPT, RPT)])
        plsc.subcore_barrier()

        def fire_idx(j, i):
            off = tile_base + j * CHUNK
            pltpu.async_copy(src_hbm.at[pl.ds(off, CHUNK)], sidx[i], isem[i])
            pltpu.async_copy(dst_hbm.at[pl.ds(off, CHUNK)], didx[i], isem[i])

        def wait_idx(i):
            pltpu.make_async_copy(src_hbm.at[pl.ds(0, CHUNK)], sidx[i],
                                  isem[i]).wait()
            pltpu.make_async_copy(src_hbm.at[pl.ds(0, CHUNK)], didx[i],
                                  isem[i]).wait()

        def fire_gather(i, p):
            pltpu.async_copy(g_hbm.at[sidx[i]], rows[p], gsem[p])

        def wait_gather(p):
            pltpu.make_async_copy(g_hbm.at[sidx[0]], rows[p], gsem[p]).wait()

        for i in range(4):
            fire_idx(i, i)
        wait_idx(0)
        fire_gather(0, 0)
        wait_idx(1)
        fire_gather(1, 1)

        @pl.loop(0, CPT // 4)
        def _(kk):
            j0 = 4 * kk
            for i in range(4):
                p = i % 2
                wait_gather(p)
                pltpu.sync_copy(rows[p], acc_sh.at[didx[i]], add=True)
                fire_idx(j0 + i + 4, i)
                wait_idx((i + 2) % 4)
                fire_gather((i + 2) % 4, p)
        wait_gather(0)
        wait_gather(1)
        wait_idx(2)
        wait_idx(3)

        plsc.subcore_barrier()
        pltpu.sync_copy(acc_sh.at[pl.ds(s * RPT, RPT)],
                        out_hbm.at[pl.ds(c * N_PAD + s * RPT, RPT)])

    return k(g, src_flat, dst_flat, zeros128)


def _tc_hw(x, W1, b1, Wc):
    BR = 1000

    def body(x_ref, w1_ref, b1_ref, wc_ref, o_ref):
        h = jnp.dot(x_ref[...], w1_ref[...],
                    preferred_element_type=jnp.float32,
                    precision=lax.Precision.HIGHEST)
        h = jnp.maximum(h + b1_ref[...], 0.0)
        o_ref[...] = jnp.dot(h, wc_ref[...],
                             preferred_element_type=jnp.float32,
                             precision=lax.Precision.HIGHEST)

    return pl.pallas_call(
        body,
        grid=(N // BR,),
        in_specs=[
            pl.BlockSpec((BR, D), lambda i: (i, 0)),
            pl.BlockSpec((D, D), lambda i: (0, 0)),
            pl.BlockSpec((1, D), lambda i: (0, 0)),
            pl.BlockSpec((D, D), lambda i: (0, 0)),
        ],
        out_specs=pl.BlockSpec((BR, D), lambda i: (i, 0)),
        out_shape=jax.ShapeDtypeStruct((N, D), jnp.float32),
    )(x, W1, b1.reshape(1, D), Wc)


def _tc_scale(deg_parts, hw):
    BR = 640

    def body(dp_ref, hw_ref, o_ref):
        i = pl.program_id(0)
        p = dp_ref[...]
        deg = 1.0 + p[0, :, 0:1] + p[1, :, 0:1]
        dinv = lax.rsqrt(deg)
        rows = i * BR + lax.broadcasted_iota(jnp.int32, (BR, 1), 0)
        o_ref[...] = jnp.where(rows < N, dinv * hw_ref[...], 0.0)

    return pl.pallas_call(
        body,
        grid=(N_PAD // BR,),
        in_specs=[
            pl.BlockSpec((NC, BR, D), lambda i: (0, i, 0)),
            pl.BlockSpec((BR, D), lambda i: (i, 0)),
        ],
        out_specs=pl.BlockSpec((BR, D), lambda i: (i, 0)),
        out_shape=jax.ShapeDtypeStruct((N_PAD, D), jnp.float32),
    )(deg_parts, hw)


def _tc_final(agg_parts, g, deg_parts, bc, Wout, bout):
    BR = 640

    def body(agg_ref, g_ref, dp_ref, bc_ref, w_ref, bo_ref, o_ref):
        p = dp_ref[...]
        deg = 1.0 + p[0, :, 0:1] + p[1, :, 0:1]
        dinv = lax.rsqrt(deg)
        ssum = agg_ref[0] + agg_ref[1] + g_ref[...]
        h2 = jnp.maximum(dinv * ssum + bc_ref[...], 0.0)
        o_ref[...] = jnp.dot(h2, w_ref[...],
                             preferred_element_type=jnp.float32,
                             precision=lax.Precision.HIGHEST) + bo_ref[...]

    return pl.pallas_call(
        body,
        grid=(N_PAD // BR,),
        in_specs=[
            pl.BlockSpec((NC, BR, D), lambda i: (0, i, 0)),
            pl.BlockSpec((BR, D), lambda i: (i, 0)),
            pl.BlockSpec((NC, BR, D), lambda i: (0, i, 0)),
            pl.BlockSpec((1, D), lambda i: (0, 0)),
            pl.BlockSpec((D, D), lambda i: (0, 0)),
            pl.BlockSpec((1, D), lambda i: (0, 0)),
        ],
        out_specs=pl.BlockSpec((BR, D), lambda i: (i, 0)),
        out_shape=jax.ShapeDtypeStruct((N, D), jnp.float32),
    )(agg_parts, g, deg_parts, bc.reshape(1, D), Wout, bout.reshape(1, D))


def kernel(x, edge_index, W1, b1, Wc, bc, Wout, bout):
    src = edge_index[0].astype(jnp.int32)
    dst = edge_index[1].astype(jnp.int32)
    n_pad_e = E_PAD - E + 4 * CHUNK
    pad = N + (jnp.arange(n_pad_e, dtype=jnp.int32) % (N_PAD - N))
    src_f = jnp.concatenate([src, pad])
    dst_f = jnp.concatenate([dst, pad])

    ones_blk = jnp.ones((CHUNK, D), jnp.float32)
    zeros128 = jnp.zeros((N_PAD, D), jnp.float32)

    deg_parts = _sc_degree(dst_f, ones_blk, zeros128).reshape(NC, N_PAD, D)
    hw = _tc_hw(x, W1, b1, Wc)
    g = _tc_scale(deg_parts, hw)
    agg_parts = _sc_aggregate(g, src_f, dst_f, zeros128).reshape(NC, N_PAD, D)
    return _tc_final(agg_parts, g, deg_parts, bc, Wout, bout)


# speedup vs baseline: 37.2371x; 1.2016x over previous
import dataclasses
import functools

import jax
import jax.numpy as jnp
from jax import lax
from jax.experimental import pallas as pl
from jax.experimental.pallas import tpu as pltpu
from jax.experimental.pallas import tpu_sc as plsc

N = 10000
E = 320000
D = 128

NC = 2
NS = 16
CHUNK = 128
CPT = 80
BCH = 20
E_PAD = NC * NS * CHUNK * CPT
N_PAD = 10240
SINK = 10200
RPT = N_PAD // NS

_MESH = dict(core_axis_name="c", subcore_axis_name="s")


TPB = CHUNK * CPT
GROUPS = TPB // 16


def _sc_degree(dst_flat):
    mesh = plsc.VectorSubcoreMesh(**_MESH)
    cp = pltpu.CompilerParams()
    if "needs_layout_passes" in pltpu.CompilerParams.__dataclass_fields__:
        cp = dataclasses.replace(cp, needs_layout_passes=False)

    @functools.partial(
        pl.kernel,
        out_type=jax.ShapeDtypeStruct((NC * N_PAD,), jnp.float32),
        compiler_params=cp,
        mesh=mesh,
        scratch_types=[
            pltpu.VMEM((TPB,), jnp.int32),
            pltpu.VMEM((N_PAD,), jnp.float32),
            pltpu.VMEM((NS, RPT), jnp.float32),
            pltpu.VMEM_SHARED((NS, NS, RPT), jnp.float32),
            pltpu.SemaphoreType.DMA,
        ],
    )
    def k(dst_hbm, out_hbm, didx_v, acc_v, red_v, part_sh, isem):
        c = lax.axis_index("c")
        s = lax.axis_index("s")
        w = c * NS + s
        pltpu.async_copy(dst_hbm.at[pl.ds(w * TPB, TPB)], didx_v, isem)

        zeros16 = jnp.zeros((16,), jnp.float32)

        @pl.loop(0, N_PAD // 16)
        def _(i):
            acc_v[pl.ds(16 * i, 16)] = zeros16

        pltpu.make_async_copy(dst_hbm.at[pl.ds(0, TPB)], didx_v, isem).wait()
        ones16 = jnp.ones((16,), jnp.float32)

        @pl.loop(0, GROUPS)
        def _(e):
            idx = didx_v[pl.ds(16 * e, 16)]
            plsc.addupdate_scatter(acc_v, [idx], ones16)

        @pl.loop(0, NS)
        def _(r):
            pltpu.sync_copy(acc_v.at[pl.ds(r * RPT, RPT)], part_sh.at[r, s])

        plsc.subcore_barrier()
        pltpu.sync_copy(part_sh.at[s], red_v)

        @pl.loop(0, RPT // 16)
        def _(gg):
            tot = red_v[0, pl.ds(16 * gg, 16)]
            for r in range(1, NS):
                tot = tot + red_v[r, pl.ds(16 * gg, 16)]
            acc_v[pl.ds(16 * gg, 16)] = tot

        pltpu.sync_copy(acc_v.at[pl.ds(0, RPT)],
                        out_hbm.at[pl.ds(c * N_PAD + s * RPT, RPT)])

    return k(dst_flat)


def _sc_aggregate(g, src_flat, dst_flat, zeros128):
    mesh = plsc.VectorSubcoreMesh(**_MESH)

    @functools.partial(
        pl.kernel,
        out_type=jax.ShapeDtypeStruct((NC * N_PAD, D), jnp.float32),
        mesh=mesh,
        scratch_types=(
            [pltpu.VMEM((CHUNK,), jnp.int32) for _ in range(8)]
            + [pltpu.VMEM((CHUNK, D), jnp.float32) for _ in range(2)]
            + [pltpu.VMEM_SHARED((N_PAD, D), jnp.float32)]
            + [pltpu.SemaphoreType.DMA for _ in range(6)]
        ),
    )
    def k(g_hbm, src_hbm, dst_hbm, zeros_hbm, out_hbm,
          s0, s1, s2, s3, d0, d1, d2, d3, rows0, rows1, acc_sh,
          gsem0, gsem1, i0, i1, i2, i3):
        sidx = [s0, s1, s2, s3]
        didx = [d0, d1, d2, d3]
        rows = [rows0, rows1]
        gsem = [gsem0, gsem1]
        isem = [i0, i1, i2, i3]
        c = lax.axis_index("c")
        s = lax.axis_index("s")
        w = c * NS + s
        tile_base = w * (CHUNK * CPT)
        pltpu.sync_copy(zeros_hbm.at[pl.ds(s * RPT, RPT)],
                        acc_sh.at[pl.ds(s * RPT, RPT)])
        plsc.subcore_barrier()

        def fire_idx(j, i):
            off = tile_base + j * CHUNK
            pltpu.async_copy(src_hbm.at[pl.ds(off, CHUNK)], sidx[i], isem[i])
            pltpu.async_copy(dst_hbm.at[pl.ds(off, CHUNK)], didx[i], isem[i])

        def wait_idx(i):
            pltpu.make_async_copy(src_hbm.at[pl.ds(0, CHUNK)], sidx[i],
                                  isem[i]).wait()
            pltpu.make_async_copy(src_hbm.at[pl.ds(0, CHUNK)], didx[i],
                                  isem[i]).wait()

        def fire_gather(i, p):
            pltpu.async_copy(g_hbm.at[sidx[i]], rows[p], gsem[p])

        def wait_gather(p):
            pltpu.make_async_copy(g_hbm.at[sidx[0]], rows[p], gsem[p]).wait()

        for i in range(4):
            fire_idx(i, i)
        wait_idx(0)
        fire_gather(0, 0)
        wait_idx(1)
        fire_gather(1, 1)

        @pl.loop(0, CPT // 4)
        def _(kk):
            j0 = 4 * kk
            for i in range(4):
                p = i % 2
                wait_gather(p)
                pltpu.sync_copy(rows[p], acc_sh.at[didx[i]], add=True)
                fire_idx(j0 + i + 4, i)
                wait_idx((i + 2) % 4)
                fire_gather((i + 2) % 4, p)
        wait_gather(0)
        wait_gather(1)
        wait_idx(2)
        wait_idx(3)

        plsc.subcore_barrier()
        pltpu.sync_copy(acc_sh.at[pl.ds(s * RPT, RPT)],
                        out_hbm.at[pl.ds(c * N_PAD + s * RPT, RPT)])

    return k(g, src_flat, dst_flat, zeros128)


def _tc_hw(x, W1, b1, Wc):
    BR = 1000

    def body(x_ref, w1_ref, b1_ref, wc_ref, o_ref):
        h = jnp.dot(x_ref[...], w1_ref[...],
                    preferred_element_type=jnp.float32,
                    precision=lax.Precision.HIGHEST)
        h = jnp.maximum(h + b1_ref[...], 0.0)
        o_ref[...] = jnp.dot(h, wc_ref[...],
                             preferred_element_type=jnp.float32,
                             precision=lax.Precision.HIGHEST)

    return pl.pallas_call(
        body,
        grid=(N // BR,),
        in_specs=[
            pl.BlockSpec((BR, D), lambda i: (i, 0)),
            pl.BlockSpec((D, D), lambda i: (0, 0)),
            pl.BlockSpec((1, D), lambda i: (0, 0)),
            pl.BlockSpec((D, D), lambda i: (0, 0)),
        ],
        out_specs=pl.BlockSpec((BR, D), lambda i: (i, 0)),
        out_shape=jax.ShapeDtypeStruct((N, D), jnp.float32),
    )(x, W1, b1.reshape(1, D), Wc)


def _tc_scale(deg0, deg1, hw):
    BR = 640

    def body(d0_ref, d1_ref, hw_ref, o_ref):
        i = pl.program_id(0)
        deg = 1.0 + d0_ref[pl.ds(i * BR, BR)] + d1_ref[pl.ds(i * BR, BR)]
        dinv = lax.rsqrt(deg).reshape(BR, 1)
        rows = i * BR + lax.broadcasted_iota(jnp.int32, (BR, 1), 0)
        o_ref[...] = jnp.where(rows < N, dinv * hw_ref[...], 0.0)

    return pl.pallas_call(
        body,
        grid=(N_PAD // BR,),
        in_specs=[
            pl.BlockSpec((N_PAD,), lambda i: (0,)),
            pl.BlockSpec((N_PAD,), lambda i: (0,)),
            pl.BlockSpec((BR, D), lambda i: (i, 0)),
        ],
        out_specs=pl.BlockSpec((BR, D), lambda i: (i, 0)),
        out_shape=jax.ShapeDtypeStruct((N_PAD, D), jnp.float32),
    )(deg0, deg1, hw)


def _tc_final(agg_parts, g, deg0, deg1, bc, Wout, bout):
    BR = 640

    def body(agg_ref, g_ref, d0_ref, d1_ref, bc_ref, w_ref, bo_ref, o_ref):
        i = pl.program_id(0)
        deg = 1.0 + d0_ref[pl.ds(i * BR, BR)] + d1_ref[pl.ds(i * BR, BR)]
        dinv = lax.rsqrt(deg).reshape(BR, 1)
        ssum = agg_ref[0] + agg_ref[1] + g_ref[...]
        h2 = jnp.maximum(dinv * ssum + bc_ref[...], 0.0)
        o_ref[...] = jnp.dot(h2, w_ref[...],
                             preferred_element_type=jnp.float32,
                             precision=lax.Precision.HIGHEST) + bo_ref[...]

    return pl.pallas_call(
        body,
        grid=(N_PAD // BR,),
        in_specs=[
            pl.BlockSpec((NC, BR, D), lambda i: (0, i, 0)),
            pl.BlockSpec((BR, D), lambda i: (i, 0)),
            pl.BlockSpec((N_PAD,), lambda i: (0,)),
            pl.BlockSpec((N_PAD,), lambda i: (0,)),
            pl.BlockSpec((1, D), lambda i: (0, 0)),
            pl.BlockSpec((D, D), lambda i: (0, 0)),
            pl.BlockSpec((1, D), lambda i: (0, 0)),
        ],
        out_specs=pl.BlockSpec((BR, D), lambda i: (i, 0)),
        out_shape=jax.ShapeDtypeStruct((N, D), jnp.float32),
    )(agg_parts, g, deg0, deg1, bc.reshape(1, D), Wout, bout.reshape(1, D))


def kernel(x, edge_index, W1, b1, Wc, bc, Wout, bout):
    src = edge_index[0].astype(jnp.int32)
    dst = edge_index[1].astype(jnp.int32)
    n_pad_e = E_PAD - E + 4 * CHUNK
    pad = N + (jnp.arange(n_pad_e, dtype=jnp.int32) % (N_PAD - N))
    src_f = jnp.concatenate([src, pad])
    dst_f = jnp.concatenate([dst, pad])

    zeros128 = jnp.zeros((N_PAD, D), jnp.float32)

    deg_flat = _sc_degree(dst_f)
    deg0, deg1 = deg_flat[:N_PAD], deg_flat[N_PAD:]
    hw = _tc_hw(x, W1, b1, Wc)
    g = _tc_scale(deg0, deg1, hw)
    agg_parts = _sc_aggregate(g, src_f, dst_f, zeros128).reshape(NC, N_PAD, D)
    return _tc_final(agg_parts, g, deg0, deg1, bc, Wout, bout)


# speedup vs baseline: 42.9098x; 1.1523x over previous
import dataclasses
import functools

import jax
import jax.numpy as jnp
from jax import lax
from jax.experimental import pallas as pl
from jax.experimental.pallas import tpu as pltpu
from jax.experimental.pallas import tpu_sc as plsc

N = 10000
E = 320000
D = 128

NC = 2
NS = 16
CHUNK = 128
TPB = E // (NC * NS)
CPT = TPB // CHUNK
TAIL = TPB - CPT * CHUNK
N_PAD = 10240
RPT = N_PAD // NS

_MESH = dict(core_axis_name="c", subcore_axis_name="s")


GROUPS = TPB // 16


def _sc_degree(dst_flat):
    mesh = plsc.VectorSubcoreMesh(**_MESH)
    cp = pltpu.CompilerParams()
    if "needs_layout_passes" in pltpu.CompilerParams.__dataclass_fields__:
        cp = dataclasses.replace(cp, needs_layout_passes=False)

    @functools.partial(
        pl.kernel,
        out_type=jax.ShapeDtypeStruct((NC * N_PAD,), jnp.float32),
        compiler_params=cp,
        mesh=mesh,
        scratch_types=[
            pltpu.VMEM((TPB,), jnp.int32),
            pltpu.VMEM((N_PAD,), jnp.float32),
            pltpu.VMEM((NS, RPT), jnp.float32),
            pltpu.VMEM_SHARED((NS, NS, RPT), jnp.float32),
            pltpu.SemaphoreType.DMA,
        ],
    )
    def k(dst_hbm, out_hbm, didx_v, acc_v, red_v, part_sh, isem):
        c = lax.axis_index("c")
        s = lax.axis_index("s")
        w = c * NS + s
        pltpu.async_copy(dst_hbm.at[pl.ds(w * TPB, TPB)], didx_v, isem)

        zeros16 = jnp.zeros((16,), jnp.float32)

        @pl.loop(0, N_PAD // 16)
        def _(i):
            acc_v[pl.ds(16 * i, 16)] = zeros16

        pltpu.make_async_copy(dst_hbm.at[pl.ds(0, TPB)], didx_v, isem).wait()
        ones16 = jnp.ones((16,), jnp.float32)

        @pl.loop(0, GROUPS)
        def _(e):
            idx = didx_v[pl.ds(16 * e, 16)]
            plsc.addupdate_scatter(acc_v, [idx], ones16)

        @pl.loop(0, NS)
        def _(r):
            pltpu.sync_copy(acc_v.at[pl.ds(r * RPT, RPT)], part_sh.at[r, s])

        plsc.subcore_barrier()
        pltpu.sync_copy(part_sh.at[s], red_v)

        @pl.loop(0, RPT // 16)
        def _(gg):
            tot = red_v[0, pl.ds(16 * gg, 16)]
            for r in range(1, NS):
                tot = tot + red_v[r, pl.ds(16 * gg, 16)]
            acc_v[pl.ds(16 * gg, 16)] = tot

        pltpu.sync_copy(acc_v.at[pl.ds(0, RPT)],
                        out_hbm.at[pl.ds(c * N_PAD + s * RPT, RPT)])

    return k(dst_flat)


def _sc_aggregate(g, src_flat, dst_flat, zeros128):
    mesh = plsc.VectorSubcoreMesh(**_MESH)

    @functools.partial(
        pl.kernel,
        out_type=jax.ShapeDtypeStruct((NC * N_PAD, D), jnp.float32),
        mesh=mesh,
        scratch_types=(
            [pltpu.VMEM((CHUNK,), jnp.int32) for _ in range(8)]
            + [pltpu.VMEM((CHUNK, D), jnp.float32) for _ in range(2)]
            + [pltpu.VMEM((TAIL,), jnp.int32), pltpu.VMEM((TAIL,), jnp.int32),
               pltpu.VMEM((TAIL, D), jnp.float32)]
            + [pltpu.VMEM_SHARED((N_PAD, D), jnp.float32)]
            + [pltpu.SemaphoreType.DMA for _ in range(6)]
        ),
    )
    def k(g_hbm, src_hbm, dst_hbm, zeros_hbm, out_hbm,
          s0, s1, s2, s3, d0, d1, d2, d3, rows0, rows1,
          tidx_s, tidx_d, trows, acc_sh,
          gsem0, gsem1, i0, i1, i2, i3):
        sidx = [s0, s1, s2, s3]
        didx = [d0, d1, d2, d3]
        rows = [rows0, rows1]
        gsem = [gsem0, gsem1]
        isem = [i0, i1, i2, i3]
        c = lax.axis_index("c")
        s = lax.axis_index("s")
        w = c * NS + s
        tile_base = w * TPB
        pltpu.sync_copy(zeros_hbm.at[pl.ds(s * RPT, RPT)],
                        acc_sh.at[pl.ds(s * RPT, RPT)])
        plsc.subcore_barrier()

        def fire_idx(j, i):
            off = jnp.minimum(tile_base + j * CHUNK, E - CHUNK)
            pltpu.async_copy(src_hbm.at[pl.ds(off, CHUNK)], sidx[i], isem[i])
            pltpu.async_copy(dst_hbm.at[pl.ds(off, CHUNK)], didx[i], isem[i])

        def wait_idx(i):
            pltpu.make_async_copy(src_hbm.at[pl.ds(0, CHUNK)], sidx[i],
                                  isem[i]).wait()
            pltpu.make_async_copy(src_hbm.at[pl.ds(0, CHUNK)], didx[i],
                                  isem[i]).wait()

        def fire_gather(i, p):
            pltpu.async_copy(g_hbm.at[sidx[i]], rows[p], gsem[p])

        def wait_gather(p):
            pltpu.make_async_copy(g_hbm.at[sidx[0]], rows[p], gsem[p]).wait()

        for i in range(4):
            fire_idx(i, i)
        wait_idx(0)
        fire_gather(0, 0)
        wait_idx(1)
        fire_gather(1, 1)

        @pl.loop(0, CPT // 4)
        def _(kk):
            j0 = 4 * kk
            for i in range(4):
                p = i % 2
                wait_gather(p)
                pltpu.sync_copy(rows[p], acc_sh.at[didx[i]], add=True)
                fire_idx(j0 + i + 4, i)
                wait_idx((i + 2) % 4)
                fire_gather((i + 2) % 4, p)

        for j in (CPT - 2, CPT - 1):
            i = j % 4
            p = j % 2
            wait_gather(p)
            pltpu.sync_copy(rows[p], acc_sh.at[didx[i]], add=True)
        wait_idx(CPT % 4)
        wait_idx((CPT + 1) % 4)
        toff = tile_base + CPT * CHUNK
        pltpu.async_copy(src_hbm.at[pl.ds(toff, TAIL)], tidx_s, isem[0])
        pltpu.async_copy(dst_hbm.at[pl.ds(toff, TAIL)], tidx_d, isem[0])
        pltpu.make_async_copy(src_hbm.at[pl.ds(0, TAIL)], tidx_s,
                              isem[0]).wait()
        pltpu.make_async_copy(src_hbm.at[pl.ds(0, TAIL)], tidx_d,
                              isem[0]).wait()
        pltpu.async_copy(g_hbm.at[tidx_s], trows, gsem[0])
        pltpu.make_async_copy(g_hbm.at[tidx_s], trows, gsem[0]).wait()
        pltpu.sync_copy(trows, acc_sh.at[tidx_d], add=True)

        plsc.subcore_barrier()
        pltpu.sync_copy(acc_sh.at[pl.ds(s * RPT, RPT)],
                        out_hbm.at[pl.ds(c * N_PAD + s * RPT, RPT)])

    return k(g, src_flat, dst_flat, zeros128)


def _tc_hw(x, W1, b1, Wc):
    BR = 1000

    def body(x_ref, w1_ref, b1_ref, wc_ref, o_ref):
        h = jnp.dot(x_ref[...], w1_ref[...],
                    preferred_element_type=jnp.float32,
                    precision=lax.Precision.DEFAULT)
        h = jnp.maximum(h + b1_ref[...], 0.0)
        o_ref[...] = jnp.dot(h, wc_ref[...],
                             preferred_element_type=jnp.float32,
                             precision=lax.Precision.DEFAULT)

    return pl.pallas_call(
        body,
        grid=(N // BR,),
        in_specs=[
            pl.BlockSpec((BR, D), lambda i: (i, 0)),
            pl.BlockSpec((D, D), lambda i: (0, 0)),
            pl.BlockSpec((1, D), lambda i: (0, 0)),
            pl.BlockSpec((D, D), lambda i: (0, 0)),
        ],
        out_specs=pl.BlockSpec((BR, D), lambda i: (i, 0)),
        out_shape=jax.ShapeDtypeStruct((N, D), jnp.float32),
    )(x, W1, b1.reshape(1, D), Wc)


def _tc_scale(deg0, deg1, hw):
    BR = 640

    def body(d0_ref, d1_ref, hw_ref, o_ref):
        i = pl.program_id(0)
        deg = 1.0 + d0_ref[pl.ds(i * BR, BR)] + d1_ref[pl.ds(i * BR, BR)]
        dinv = lax.rsqrt(deg).reshape(BR, 1)
        o_ref[...] = dinv * hw_ref[...]

    return pl.pallas_call(
        body,
        grid=(N_PAD // BR,),
        in_specs=[
            pl.BlockSpec((N_PAD,), lambda i: (0,)),
            pl.BlockSpec((N_PAD,), lambda i: (0,)),
            pl.BlockSpec((BR, D), lambda i: (i, 0)),
        ],
        out_specs=pl.BlockSpec((BR, D), lambda i: (i, 0)),
        out_shape=jax.ShapeDtypeStruct((N, D), jnp.float32),
    )(deg0, deg1, hw)


def _tc_final(agg_parts, g, deg0, deg1, bc, Wout, bout):
    BR = 640

    def body(agg_ref, g_ref, d0_ref, d1_ref, bc_ref, w_ref, bo_ref, o_ref):
        i = pl.program_id(0)
        deg = 1.0 + d0_ref[pl.ds(i * BR, BR)] + d1_ref[pl.ds(i * BR, BR)]
        dinv = lax.rsqrt(deg).reshape(BR, 1)
        ssum = agg_ref[0] + agg_ref[1] + g_ref[...]
        h2 = jnp.maximum(dinv * ssum + bc_ref[...], 0.0)
        o_ref[...] = jnp.dot(h2, w_ref[...],
                             preferred_element_type=jnp.float32,
                             precision=lax.Precision.DEFAULT) + bo_ref[...]

    return pl.pallas_call(
        body,
        grid=(N_PAD // BR,),
        in_specs=[
            pl.BlockSpec((NC, BR, D), lambda i: (0, i, 0)),
            pl.BlockSpec((BR, D), lambda i: (i, 0)),
            pl.BlockSpec((N_PAD,), lambda i: (0,)),
            pl.BlockSpec((N_PAD,), lambda i: (0,)),
            pl.BlockSpec((1, D), lambda i: (0, 0)),
            pl.BlockSpec((D, D), lambda i: (0, 0)),
            pl.BlockSpec((1, D), lambda i: (0, 0)),
        ],
        out_specs=pl.BlockSpec((BR, D), lambda i: (i, 0)),
        out_shape=jax.ShapeDtypeStruct((N, D), jnp.float32),
    )(agg_parts, g, deg0, deg1, bc.reshape(1, D), Wout, bout.reshape(1, D))


def kernel(x, edge_index, W1, b1, Wc, bc, Wout, bout):
    src = edge_index[0].astype(jnp.int32)
    dst = edge_index[1].astype(jnp.int32)
    zeros128 = jnp.zeros((N_PAD, D), jnp.float32)

    deg_flat = _sc_degree(dst)
    deg0, deg1 = deg_flat[:N_PAD], deg_flat[N_PAD:]
    hw = _tc_hw(x, W1, b1, Wc)
    g = _tc_scale(deg0, deg1, hw)
    agg_parts = _sc_aggregate(g, src, dst, zeros128).reshape(NC, N_PAD, D)
    return _tc_final(agg_parts, g, deg0, deg1, bc, Wout, bout)


# speedup vs baseline: 46.4967x; 1.0836x over previous
import dataclasses
import functools

import jax
import jax.numpy as jnp
from jax import lax
from jax.experimental import pallas as pl
from jax.experimental.pallas import tpu as pltpu
from jax.experimental.pallas import tpu_sc as plsc

N = 10000
E = 320000
D = 128

NC = 2
NS = 16
CHUNK = 128
NCH = E // CHUNK
CPT = NCH // (NC * NS)
XTRA = NCH - CPT * NC * NS
TPB = CPT * CHUNK
N_PAD = 10240
RPT = N_PAD // NS

_MESH = dict(core_axis_name="c", subcore_axis_name="s")


GROUPS = TPB // 16


def _sc_degree(edge_index):
    mesh = plsc.VectorSubcoreMesh(**_MESH)
    cp = pltpu.CompilerParams()
    if "needs_layout_passes" in pltpu.CompilerParams.__dataclass_fields__:
        cp = dataclasses.replace(cp, needs_layout_passes=False)

    @functools.partial(
        pl.kernel,
        out_type=jax.ShapeDtypeStruct((NC * N_PAD,), jnp.float32),
        compiler_params=cp,
        mesh=mesh,
        scratch_types=[
            pltpu.VMEM((2, TPB), jnp.int32),
            pltpu.VMEM((2, CHUNK), jnp.int32),
            pltpu.VMEM((N_PAD,), jnp.float32),
            pltpu.VMEM((NS, RPT), jnp.float32),
            pltpu.VMEM_SHARED((NS, NS, RPT), jnp.float32),
            pltpu.SemaphoreType.DMA,
        ],
    )
    def k(ei_hbm, out_hbm, pidx_v, epair_v, acc_v, red_v, part_sh, isem):
        c = lax.axis_index("c")
        s = lax.axis_index("s")
        w = c * NS + s
        pltpu.async_copy(ei_hbm.at[:, pl.ds(w * TPB, TPB)], pidx_v, isem)

        @pl.when(w < XTRA)
        def _():
            pltpu.async_copy(ei_hbm.at[:, pl.ds((CPT * NC * NS + w) * CHUNK,
                                                CHUNK)], epair_v, isem)

        zeros16 = jnp.zeros((16,), jnp.float32)

        @pl.loop(0, N_PAD // 16)
        def _(i):
            acc_v[pl.ds(16 * i, 16)] = zeros16

        pltpu.make_async_copy(ei_hbm.at[:, pl.ds(0, TPB)], pidx_v,
                              isem).wait()
        ones16 = jnp.ones((16,), jnp.float32)

        @pl.loop(0, GROUPS)
        def _(e):
            idx = pidx_v[1, pl.ds(16 * e, 16)]
            plsc.addupdate_scatter(acc_v, [idx], ones16)

        @pl.when(w < XTRA)
        def _():
            pltpu.make_async_copy(ei_hbm.at[:, pl.ds(0, CHUNK)], epair_v,
                                  isem).wait()

            @pl.loop(0, CHUNK // 16)
            def _(e):
                idx = epair_v[1, pl.ds(16 * e, 16)]
                plsc.addupdate_scatter(acc_v, [idx], ones16)

        @pl.loop(0, NS)
        def _(r):
            pltpu.sync_copy(acc_v.at[pl.ds(r * RPT, RPT)], part_sh.at[r, s])

        plsc.subcore_barrier()
        pltpu.sync_copy(part_sh.at[s], red_v)

        @pl.loop(0, RPT // 16)
        def _(gg):
            tot = red_v[0, pl.ds(16 * gg, 16)]
            for r in range(1, NS):
                tot = tot + red_v[r, pl.ds(16 * gg, 16)]
            acc_v[pl.ds(16 * gg, 16)] = tot

        pltpu.sync_copy(acc_v.at[pl.ds(0, RPT)],
                        out_hbm.at[pl.ds(c * N_PAD + s * RPT, RPT)])

    return k(edge_index)


def _sc_aggregate(g, edge_index, zeros128):
    mesh = plsc.VectorSubcoreMesh(**_MESH)

    @functools.partial(
        pl.kernel,
        out_type=jax.ShapeDtypeStruct((NC * N_PAD, D), jnp.float32),
        mesh=mesh,
        scratch_types=(
            [pltpu.VMEM((2, CHUNK), jnp.int32) for _ in range(4)]
            + [pltpu.VMEM((CHUNK, D), jnp.float32) for _ in range(2)]
            + [pltpu.VMEM_SHARED((N_PAD, D), jnp.float32)]
            + [pltpu.SemaphoreType.DMA for _ in range(6)]
        ),
    )
    def k(g_hbm, ei_hbm, zeros_hbm, out_hbm,
          p0, p1, p2, p3, rows0, rows1, acc_sh,
          gsem0, gsem1, i0, i1, i2, i3):
        pair = [p0, p1, p2, p3]
        sidx = [pr.at[0] for pr in pair]
        didx = [pr.at[1] for pr in pair]
        rows = [rows0, rows1]
        gsem = [gsem0, gsem1]
        isem = [i0, i1, i2, i3]
        c = lax.axis_index("c")
        s = lax.axis_index("s")
        w = c * NS + s
        tile_base = w * TPB
        pltpu.sync_copy(zeros_hbm.at[pl.ds(s * RPT, RPT)],
                        acc_sh.at[pl.ds(s * RPT, RPT)])
        plsc.subcore_barrier()

        def fire_idx(j, i):
            off = jnp.minimum(tile_base + j * CHUNK, E - CHUNK)
            pltpu.async_copy(ei_hbm.at[:, pl.ds(off, CHUNK)], pair[i],
                             isem[i])

        def wait_idx(i):
            pltpu.make_async_copy(ei_hbm.at[:, pl.ds(0, CHUNK)], pair[i],
                                  isem[i]).wait()

        def fire_gather(i, p):
            pltpu.async_copy(g_hbm.at[sidx[i]], rows[p], gsem[p])

        def wait_gather(p):
            pltpu.make_async_copy(g_hbm.at[sidx[0]], rows[p], gsem[p]).wait()

        for i in range(4):
            fire_idx(i, i)
        wait_idx(0)
        fire_gather(0, 0)
        wait_idx(1)
        fire_gather(1, 1)

        @pl.loop(0, CPT // 4)
        def _(kk):
            j0 = 4 * kk
            for i in range(4):
                p = i % 2
                wait_gather(p)
                pltpu.sync_copy(rows[p], acc_sh.at[didx[i]], add=True)
                fire_idx(j0 + i + 4, i)
                wait_idx((i + 2) % 4)
                fire_gather((i + 2) % 4, p)

        for j in (CPT - 2, CPT - 1):
            i = j % 4
            p = j % 2
            wait_gather(p)
            pltpu.sync_copy(rows[p], acc_sh.at[didx[i]], add=True)
        wait_idx(CPT % 4)
        wait_idx((CPT + 1) % 4)
        @pl.when(w < XTRA)
        def _():
            toff = (CPT * NC * NS + w) * CHUNK
            pltpu.async_copy(ei_hbm.at[:, pl.ds(toff, CHUNK)], pair[0],
                             isem[0])
            pltpu.make_async_copy(ei_hbm.at[:, pl.ds(0, CHUNK)], pair[0],
                                  isem[0]).wait()
            pltpu.async_copy(g_hbm.at[sidx[0]], rows[0], gsem[0])
            pltpu.make_async_copy(g_hbm.at[sidx[0]], rows[0],
                                  gsem[0]).wait()
            pltpu.sync_copy(rows[0], acc_sh.at[didx[0]], add=True)

        plsc.subcore_barrier()
        pltpu.sync_copy(acc_sh.at[pl.ds(s * RPT, RPT)],
                        out_hbm.at[pl.ds(c * N_PAD + s * RPT, RPT)])

    return k(g, edge_index, zeros128)


def _tc_hw(x, W1, b1, Wc):
    BR = 1000

    def body(x_ref, w1_ref, b1_ref, wc_ref, o_ref):
        h = jnp.dot(x_ref[...], w1_ref[...],
                    preferred_element_type=jnp.float32,
                    precision=lax.Precision.DEFAULT)
        h = jnp.maximum(h + b1_ref[...], 0.0)
        o_ref[...] = jnp.dot(h, wc_ref[...],
                             preferred_element_type=jnp.float32,
                             precision=lax.Precision.DEFAULT)

    return pl.pallas_call(
        body,
        grid=(N // BR,),
        in_specs=[
            pl.BlockSpec((BR, D), lambda i: (i, 0)),
            pl.BlockSpec((D, D), lambda i: (0, 0)),
            pl.BlockSpec((1, D), lambda i: (0, 0)),
            pl.BlockSpec((D, D), lambda i: (0, 0)),
        ],
        out_specs=pl.BlockSpec((BR, D), lambda i: (i, 0)),
        out_shape=jax.ShapeDtypeStruct((N, D), jnp.float32),
    )(x, W1, b1.reshape(1, D), Wc)


def _tc_scale(deg0, deg1, hw):
    BR = 640

    def body(d0_ref, d1_ref, hw_ref, o_ref):
        i = pl.program_id(0)
        deg = 1.0 + d0_ref[pl.ds(i * BR, BR)] + d1_ref[pl.ds(i * BR, BR)]
        dinv = lax.rsqrt(deg).reshape(BR, 1)
        o_ref[...] = dinv * hw_ref[...]

    return pl.pallas_call(
        body,
        grid=(N_PAD // BR,),
        in_specs=[
            pl.BlockSpec((N_PAD,), lambda i: (0,)),
            pl.BlockSpec((N_PAD,), lambda i: (0,)),
            pl.BlockSpec((BR, D), lambda i: (i, 0)),
        ],
        out_specs=pl.BlockSpec((BR, D), lambda i: (i, 0)),
        out_shape=jax.ShapeDtypeStruct((N, D), jnp.float32),
    )(deg0, deg1, hw)


def _tc_final(agg_parts, g, deg0, deg1, bc, Wout, bout):
    BR = 640

    def body(agg_ref, g_ref, d0_ref, d1_ref, bc_ref, w_ref, bo_ref, o_ref):
        i = pl.program_id(0)
        deg = 1.0 + d0_ref[pl.ds(i * BR, BR)] + d1_ref[pl.ds(i * BR, BR)]
        dinv = lax.rsqrt(deg).reshape(BR, 1)
        ssum = agg_ref[0] + agg_ref[1] + g_ref[...]
        h2 = jnp.maximum(dinv * ssum + bc_ref[...], 0.0)
        o_ref[...] = jnp.dot(h2, w_ref[...],
                             preferred_element_type=jnp.float32,
                             precision=lax.Precision.DEFAULT) + bo_ref[...]

    return pl.pallas_call(
        body,
        grid=(N_PAD // BR,),
        in_specs=[
            pl.BlockSpec((NC, BR, D), lambda i: (0, i, 0)),
            pl.BlockSpec((BR, D), lambda i: (i, 0)),
            pl.BlockSpec((N_PAD,), lambda i: (0,)),
            pl.BlockSpec((N_PAD,), lambda i: (0,)),
            pl.BlockSpec((1, D), lambda i: (0, 0)),
            pl.BlockSpec((D, D), lambda i: (0, 0)),
            pl.BlockSpec((1, D), lambda i: (0, 0)),
        ],
        out_specs=pl.BlockSpec((BR, D), lambda i: (i, 0)),
        out_shape=jax.ShapeDtypeStruct((N, D), jnp.float32),
    )(agg_parts, g, deg0, deg1, bc.reshape(1, D), Wout, bout.reshape(1, D))


def kernel(x, edge_index, W1, b1, Wc, bc, Wout, bout):
    ei = edge_index.astype(jnp.int32)
    zeros128 = jnp.zeros((N_PAD, D), jnp.float32)

    deg_flat = _sc_degree(ei)
    deg0, deg1 = deg_flat[:N_PAD], deg_flat[N_PAD:]
    hw = _tc_hw(x, W1, b1, Wc)
    g = _tc_scale(deg0, deg1, hw)
    agg_parts = _sc_aggregate(g, ei, zeros128).reshape(NC, N_PAD, D)
    return _tc_final(agg_parts, g, deg0, deg1, bc, Wout, bout)


# speedup vs baseline: 50.0596x; 1.0766x over previous
import dataclasses
import functools

import jax
import jax.numpy as jnp
from jax import lax
from jax.experimental import pallas as pl
from jax.experimental.pallas import tpu as pltpu
from jax.experimental.pallas import tpu_sc as plsc

N = 10000
E = 320000
D = 128

NC = 2
NS = 16
CHUNK = 128
NCH = E // CHUNK
CPT = NCH // (NC * NS)
XTRA = NCH - CPT * NC * NS
TPB = CPT * CHUNK
N_PAD = 10240
RPT = N_PAD // NS

_MESH = dict(core_axis_name="c", subcore_axis_name="s")


GROUPS = TPB // 16


def _sc_degree(edge_index):
    mesh = plsc.VectorSubcoreMesh(**_MESH)
    cp = pltpu.CompilerParams()
    if "needs_layout_passes" in pltpu.CompilerParams.__dataclass_fields__:
        cp = dataclasses.replace(cp, needs_layout_passes=False)

    @functools.partial(
        pl.kernel,
        out_type=jax.ShapeDtypeStruct((NC * N_PAD,), jnp.float32),
        compiler_params=cp,
        mesh=mesh,
        scratch_types=[
            pltpu.VMEM((2, TPB), jnp.int32),
            pltpu.VMEM((2, CHUNK), jnp.int32),
            pltpu.VMEM((N_PAD,), jnp.float32),
            pltpu.VMEM((NS, RPT), jnp.float32),
            pltpu.VMEM_SHARED((NS, NS, RPT), jnp.float32),
            pltpu.SemaphoreType.DMA,
        ],
    )
    def k(ei_hbm, out_hbm, pidx_v, epair_v, acc_v, red_v, part_sh, isem):
        c = lax.axis_index("c")
        s = lax.axis_index("s")
        w = c * NS + s
        pltpu.async_copy(ei_hbm.at[:, pl.ds(w * TPB, TPB)], pidx_v, isem)

        @pl.when(w < XTRA)
        def _():
            pltpu.async_copy(ei_hbm.at[:, pl.ds((CPT * NC * NS + w) * CHUNK,
                                                CHUNK)], epair_v, isem)

        zeros16 = jnp.zeros((16,), jnp.float32)

        @pl.loop(0, N_PAD // 16)
        def _(i):
            acc_v[pl.ds(16 * i, 16)] = zeros16

        pltpu.make_async_copy(ei_hbm.at[:, pl.ds(0, TPB)], pidx_v,
                              isem).wait()
        ones16 = jnp.ones((16,), jnp.float32)

        @pl.loop(0, GROUPS)
        def _(e):
            idx = pidx_v[1, pl.ds(16 * e, 16)]
            plsc.addupdate_scatter(acc_v, [idx], ones16)

        @pl.when(w < XTRA)
        def _():
            pltpu.make_async_copy(ei_hbm.at[:, pl.ds(0, CHUNK)], epair_v,
                                  isem).wait()

            @pl.loop(0, CHUNK // 16)
            def _(e):
                idx = epair_v[1, pl.ds(16 * e, 16)]
                plsc.addupdate_scatter(acc_v, [idx], ones16)

        @pl.loop(0, NS)
        def _(r):
            pltpu.sync_copy(acc_v.at[pl.ds(r * RPT, RPT)], part_sh.at[r, s])

        plsc.subcore_barrier()
        pltpu.sync_copy(part_sh.at[s], red_v)

        @pl.loop(0, RPT // 16)
        def _(gg):
            tot = red_v[0, pl.ds(16 * gg, 16)]
            for r in range(1, NS):
                tot = tot + red_v[r, pl.ds(16 * gg, 16)]
            acc_v[pl.ds(16 * gg, 16)] = tot

        pltpu.sync_copy(acc_v.at[pl.ds(0, RPT)],
                        out_hbm.at[pl.ds(c * N_PAD + s * RPT, RPT)])

    return k(edge_index)


def _sc_aggregate(g, edge_index):
    mesh = plsc.VectorSubcoreMesh(**_MESH)

    @functools.partial(
        pl.kernel,
        out_type=jax.ShapeDtypeStruct((NC * N_PAD, D), jnp.float32),
        mesh=mesh,
        scratch_types=(
            [pltpu.VMEM((2, CHUNK), jnp.int32) for _ in range(4)]
            + [pltpu.VMEM((CHUNK, D), jnp.float32) for _ in range(2)]
            + [pltpu.VMEM_SHARED((N_PAD, D), jnp.float32)]
            + [pltpu.SemaphoreType.DMA for _ in range(6)]
        ),
    )
    def k(g_hbm, ei_hbm, out_hbm,
          p0, p1, p2, p3, rows0, rows1, acc_sh,
          gsem0, gsem1, i0, i1, i2, i3):
        pair = [p0, p1, p2, p3]
        sidx = [pr.at[0] for pr in pair]
        didx = [pr.at[1] for pr in pair]
        rows = [rows0, rows1]
        gsem = [gsem0, gsem1]
        isem = [i0, i1, i2, i3]
        c = lax.axis_index("c")
        s = lax.axis_index("s")
        w = c * NS + s
        tile_base = w * TPB
        zeros16 = jnp.zeros((16,), jnp.float32)

        @pl.loop(0, CHUNK)
        def _(z):
            for col in range(D // 16):
                rows0[z, pl.ds(16 * col, 16)] = zeros16

        for r5 in range(RPT // CHUNK):
            pltpu.sync_copy(rows0,
                            acc_sh.at[pl.ds(s * RPT + r5 * CHUNK, CHUNK)])
        plsc.subcore_barrier()

        def fire_idx(j, i):
            off = jnp.minimum(tile_base + j * CHUNK, E - CHUNK)
            pltpu.async_copy(ei_hbm.at[:, pl.ds(off, CHUNK)], pair[i],
                             isem[i])

        def wait_idx(i):
            pltpu.make_async_copy(ei_hbm.at[:, pl.ds(0, CHUNK)], pair[i],
                                  isem[i]).wait()

        def fire_gather(i, p):
            pltpu.async_copy(g_hbm.at[sidx[i]], rows[p], gsem[p])

        def wait_gather(p):
            pltpu.make_async_copy(g_hbm.at[sidx[0]], rows[p], gsem[p]).wait()

        for i in range(4):
            fire_idx(i, i)
        wait_idx(0)
        fire_gather(0, 0)
        wait_idx(1)
        fire_gather(1, 1)

        @pl.loop(0, CPT // 4)
        def _(kk):
            j0 = 4 * kk
            for i in range(4):
                p = i % 2
                wait_gather(p)
                pltpu.sync_copy(rows[p], acc_sh.at[didx[i]], add=True)
                fire_idx(j0 + i + 4, i)
                wait_idx((i + 2) % 4)
                fire_gather((i + 2) % 4, p)

        for j in (CPT - 2, CPT - 1):
            i = j % 4
            p = j % 2
            wait_gather(p)
            pltpu.sync_copy(rows[p], acc_sh.at[didx[i]], add=True)
        wait_idx(CPT % 4)
        wait_idx((CPT + 1) % 4)
        @pl.when(w < XTRA)
        def _():
            toff = (CPT * NC * NS + w) * CHUNK
            pltpu.async_copy(ei_hbm.at[:, pl.ds(toff, CHUNK)], pair[0],
                             isem[0])
            pltpu.make_async_copy(ei_hbm.at[:, pl.ds(0, CHUNK)], pair[0],
                                  isem[0]).wait()
            pltpu.async_copy(g_hbm.at[sidx[0]], rows[0], gsem[0])
            pltpu.make_async_copy(g_hbm.at[sidx[0]], rows[0],
                                  gsem[0]).wait()
            pltpu.sync_copy(rows[0], acc_sh.at[didx[0]], add=True)

        plsc.subcore_barrier()
        pltpu.sync_copy(acc_sh.at[pl.ds(s * RPT, RPT)],
                        out_hbm.at[pl.ds(c * N_PAD + s * RPT, RPT)])

    return k(g, edge_index)


def _tc_hw(x, W1, b1, Wc):
    BR = 2000

    def body(x_ref, w1_ref, b1_ref, wc_ref, o_ref):
        h = jnp.dot(x_ref[...], w1_ref[...],
                    preferred_element_type=jnp.float32,
                    precision=lax.Precision.DEFAULT)
        h = jnp.maximum(h + b1_ref[...], 0.0)
        o_ref[...] = jnp.dot(h, wc_ref[...],
                             preferred_element_type=jnp.float32,
                             precision=lax.Precision.DEFAULT)

    return pl.pallas_call(
        body,
        grid=(N // BR,),
        in_specs=[
            pl.BlockSpec((BR, D), lambda i: (i, 0)),
            pl.BlockSpec((D, D), lambda i: (0, 0)),
            pl.BlockSpec((1, D), lambda i: (0, 0)),
            pl.BlockSpec((D, D), lambda i: (0, 0)),
        ],
        out_specs=pl.BlockSpec((BR, D), lambda i: (i, 0)),
        out_shape=jax.ShapeDtypeStruct((N, D), jnp.float32),
    )(x, W1, b1.reshape(1, D), Wc)


def _tc_scale(deg0, deg1, hw):
    BR = 1280

    def body(d0_ref, d1_ref, hw_ref, o_ref):
        i = pl.program_id(0)
        deg = 1.0 + d0_ref[pl.ds(i * BR, BR)] + d1_ref[pl.ds(i * BR, BR)]
        dinv = lax.rsqrt(deg).reshape(BR, 1)
        o_ref[...] = dinv * hw_ref[...]

    return pl.pallas_call(
        body,
        grid=(N_PAD // BR,),
        in_specs=[
            pl.BlockSpec((N_PAD,), lambda i: (0,)),
            pl.BlockSpec((N_PAD,), lambda i: (0,)),
            pl.BlockSpec((BR, D), lambda i: (i, 0)),
        ],
        out_specs=pl.BlockSpec((BR, D), lambda i: (i, 0)),
        out_shape=jax.ShapeDtypeStruct((N, D), jnp.float32),
    )(deg0, deg1, hw)


def _tc_final(agg_parts, g, deg0, deg1, bc, Wout, bout):
    BR = 1280

    def body(agg_ref, g_ref, d0_ref, d1_ref, bc_ref, w_ref, bo_ref, o_ref):
        i = pl.program_id(0)
        deg = 1.0 + d0_ref[pl.ds(i * BR, BR)] + d1_ref[pl.ds(i * BR, BR)]
        dinv = lax.rsqrt(deg).reshape(BR, 1)
        ssum = agg_ref[0] + agg_ref[1] + g_ref[...]
        h2 = jnp.maximum(dinv * ssum + bc_ref[...], 0.0)
        o_ref[...] = jnp.dot(h2, w_ref[...],
                             preferred_element_type=jnp.float32,
                             precision=lax.Precision.DEFAULT) + bo_ref[...]

    return pl.pallas_call(
        body,
        grid=(N_PAD // BR,),
        in_specs=[
            pl.BlockSpec((NC, BR, D), lambda i: (0, i, 0)),
            pl.BlockSpec((BR, D), lambda i: (i, 0)),
            pl.BlockSpec((N_PAD,), lambda i: (0,)),
            pl.BlockSpec((N_PAD,), lambda i: (0,)),
            pl.BlockSpec((1, D), lambda i: (0, 0)),
            pl.BlockSpec((D, D), lambda i: (0, 0)),
            pl.BlockSpec((1, D), lambda i: (0, 0)),
        ],
        out_specs=pl.BlockSpec((BR, D), lambda i: (i, 0)),
        out_shape=jax.ShapeDtypeStruct((N, D), jnp.float32),
    )(agg_parts, g, deg0, deg1, bc.reshape(1, D), Wout, bout.reshape(1, D))


def kernel(x, edge_index, W1, b1, Wc, bc, Wout, bout):
    ei = edge_index.astype(jnp.int32)

    deg_flat = _sc_degree(ei)
    deg0, deg1 = deg_flat[:N_PAD], deg_flat[N_PAD:]
    hw = _tc_hw(x, W1, b1, Wc)
    g = _tc_scale(deg0, deg1, hw)
    agg_parts = _sc_aggregate(g, ei).reshape(NC, N_PAD, D)
    return _tc_final(agg_parts, g, deg0, deg1, bc, Wout, bout)
